# baseline (speedup 1.0000x reference)
import functools

import jax
import jax.numpy as jnp
from jax import lax
from jax.experimental import pallas as pl
from jax.experimental.pallas import tpu as pltpu
from jax.experimental.pallas import tpu_sc as plsc

N_NODES = 10000
N_EDGES = 320000
D = 128

NC = 2
NS = 16
NW = NC * NS
EPW = N_EDGES // NW
CHUNK = 40
EPW_PAD = 10080
NCHUNK = EPW_PAD // CHUNK
ACC_ROWS = N_NODES + 32
RPT = 624
RTAIL = N_NODES - NS * RPT

_ROW_BLK = 1000


NBUF = 7
NBUF_I = 14
LOOK = 5
LOOK_I = 9
UNROLL = 14


def _sc_seg_body(h_hbm, idx_hbm, z_hbm, out_hbm,
                 idx_v, rows_v, sem_i, sem_g, sem_s, acc):
    c = lax.axis_index("c")
    s = lax.axis_index("s")
    wid = s * NC + c

    def rows(b):
        return rows_v.at[pl.ds(b * CHUNK, CHUNK)]

    def start_idx(j, bi):
        pltpu.async_copy(idx_hbm.at[wid, j], idx_v.at[bi], sem_i.at[bi])

    def wait_idx(j, bi):
        pltpu.make_async_copy(idx_hbm.at[wid, j], idx_v.at[bi],
                              sem_i.at[bi]).wait()

    def start_gather(j, b, bi):
        pltpu.async_copy(h_hbm.at[idx_v.at[bi, 0]], rows(b), sem_g.at[b])

    def wait_gather(j, b, bi):
        pltpu.make_async_copy(h_hbm.at[idx_v.at[bi, 0]], rows(b),
                              sem_g.at[b]).wait()

    def start_scatter(j, b, bi):
        pltpu.async_copy(rows(b), acc.at[idx_v.at[bi, 1]], sem_s.at[b],
                         add=True)

    def wait_scatter(j, b, bi):
        pltpu.make_async_copy(rows(b), acc.at[idx_v.at[bi, 1]],
                              sem_s.at[b]).wait()

    pltpu.sync_copy(z_hbm.at[pl.ds(s * RPT, RPT)],
                    acc.at[pl.ds(s * RPT, RPT)])

    @pl.when(s == 0)
    def _():
        pltpu.sync_copy(z_hbm.at[pl.ds(NS * RPT, RTAIL)],
                        acc.at[pl.ds(NS * RPT, RTAIL)])
    plsc.subcore_barrier()

    for j in range(LOOK_I):
        start_idx(j, j % NBUF_I)
    for j in range(LOOK):
        wait_idx(j, j % NBUF_I)
        start_gather(j, j % NBUF, j % NBUF_I)

    def outer(o, carry):
        j0 = o * UNROLL
        for u in range(UNROLL):
            j = j0 + u
            b, bi = u % NBUF, u % NBUF_I
            bg, big = (u + LOOK) % NBUF, (u + LOOK) % NBUF_I
            ji, jg = j + LOOK_I, j + LOOK

            @pl.when(ji < NCHUNK)
            def _():
                start_idx(ji, (u + LOOK_I) % NBUF_I)

            @pl.when(jg < NCHUNK)
            def _():
                @pl.when(jg >= NBUF)
                def _():
                    wait_scatter(jg - NBUF, bg, (u + LOOK - NBUF) % NBUF_I)
                wait_idx(jg, big)
                start_gather(jg, bg, big)

            wait_gather(j, b, bi)
            start_scatter(j, b, bi)
        return carry

    lax.fori_loop(0, NCHUNK // UNROLL, outer, 0)
    for j in range(NCHUNK - NBUF, NCHUNK):
        wait_scatter(j, j % NBUF, j % NBUF_I)
    plsc.subcore_barrier()
    pltpu.sync_copy(acc.at[pl.ds(s * RPT, RPT)],
                    out_hbm.at[c, pl.ds(s * RPT, RPT)])

    @pl.when(s == 0)
    def _():
        pltpu.sync_copy(acc.at[pl.ds(NS * RPT, RTAIL)],
                        out_hbm.at[c, pl.ds(NS * RPT, RTAIL)])


def _sc_segment_sum(h, idx_packed, zeros):
    mesh = plsc.VectorSubcoreMesh(core_axis_name="c", subcore_axis_name="s")
    k = pl.kernel(
        _sc_seg_body,
        out_type=jax.ShapeDtypeStruct((NC, N_NODES, D), jnp.float32),
        mesh=mesh,
        scratch_types=[
            pltpu.VMEM((NBUF_I, 2, CHUNK), jnp.int32),
            pltpu.VMEM((NBUF * CHUNK, D), jnp.float32),
            pltpu.SemaphoreType.DMA((NBUF_I,)),
            pltpu.SemaphoreType.DMA((NBUF,)),
            pltpu.SemaphoreType.DMA((NBUF,)),
            pltpu.VMEM_SHARED((ACC_ROWS, D), jnp.float32),
        ],
    )
    return k(h, idx_packed, zeros)


_HSPEC = pl.BlockSpec((_ROW_BLK, D), lambda i: (i, 0))
_WSPEC = pl.BlockSpec((D, D), lambda i: (0, 0))
_BSPEC = pl.BlockSpec((1, D), lambda i: (0, 0))
_PSPEC = pl.BlockSpec((NC, _ROW_BLK, D), lambda i: (0, i, 0))
_2OUT = [jax.ShapeDtypeStruct((N_NODES, D), jnp.float32)] * 2


def _in_body(x_ref, w_ref, b_ref, ow_ref, h_ref, oacc_ref):
    h = (jnp.dot(x_ref[...], w_ref[...],
                 preferred_element_type=jnp.float32) + b_ref[...])
    h_ref[...] = h
    oacc_ref[...] = jnp.dot(h, ow_ref[...], preferred_element_type=jnp.float32)


def _tc_in(x, W, b, oW):
    return pl.pallas_call(
        _in_body,
        grid=(N_NODES // _ROW_BLK,),
        in_specs=[_HSPEC, _WSPEC, _BSPEC, _WSPEC],
        out_specs=[_HSPEC, _HSPEC],
        out_shape=_2OUT,
    )(x, W, b, oW)


def _mid_body(h_ref, p_ref, w_ref, b_ref, ow_ref, oacc_in_ref,
              h_ref_o, oacc_ref):
    acc = h_ref[...] + p_ref[0] + p_ref[1]
    hn = (jnp.dot(acc, w_ref[...],
                  preferred_element_type=jnp.float32) + b_ref[...])
    h_ref_o[...] = hn
    oacc_ref[...] = oacc_in_ref[...] + jnp.dot(
        hn, ow_ref[...], preferred_element_type=jnp.float32)


def _tc_mid(h, p, W, b, oW, oacc):
    return pl.pallas_call(
        _mid_body,
        grid=(N_NODES // _ROW_BLK,),
        in_specs=[_HSPEC, _PSPEC, _WSPEC, _BSPEC, _WSPEC, _HSPEC],
        out_specs=[_HSPEC, _HSPEC],
        out_shape=_2OUT,
    )(h, p, W, b, oW, oacc)


def _fin_body(h_ref, p_ref, w_ref, b_ref, ow_ref, oacc_in_ref, ob_ref, o_ref):
    acc = h_ref[...] + p_ref[0] + p_ref[1]
    hn = (jnp.dot(acc, w_ref[...],
                  preferred_element_type=jnp.float32) + b_ref[...])
    o_ref[...] = (oacc_in_ref[...] + jnp.dot(
        hn, ow_ref[...], preferred_element_type=jnp.float32) + ob_ref[...])


def _tc_fin(h, p, W, b, oW, oacc, ob):
    return pl.pallas_call(
        _fin_body,
        grid=(N_NODES // _ROW_BLK,),
        in_specs=[_HSPEC, _PSPEC, _WSPEC, _BSPEC, _WSPEC, _HSPEC, _BSPEC],
        out_specs=_HSPEC,
        out_shape=jax.ShapeDtypeStruct((N_NODES, D), jnp.float32),
    )(h, p, W, b, oW, oacc, ob)


def kernel(x, edge_index, in_W, in_b, W0, b0, W1, b1, W2, b2, out_W, out_b):
    src = edge_index[0].astype(jnp.int32).reshape(NW, EPW)
    dst = edge_index[1].astype(jnp.int32).reshape(NW, EPW)
    npad = EPW_PAD - EPW
    pad_src = jnp.broadcast_to((jnp.arange(npad, dtype=jnp.int32) * 125)
                               % N_NODES, (NW, npad))
    pad_dst = jnp.broadcast_to(N_NODES + (jnp.arange(npad, dtype=jnp.int32)
                                          % 32), (NW, npad))
    src = jnp.concatenate([src, pad_src], axis=1)
    dst = jnp.concatenate([dst, pad_dst], axis=1)
    src = src.reshape(NW, NCHUNK, 1, CHUNK)
    dst = dst.reshape(NW, NCHUNK, 1, CHUNK)
    idx_packed = jnp.concatenate([src, dst], axis=2)
    zeros = jnp.zeros((N_NODES, D), jnp.float32)
    oW = [out_W[i * D:(i + 1) * D] for i in range(4)]

    h, oacc = _tc_in(x, in_W, in_b.reshape(1, D), oW[0])
    p = _sc_segment_sum(h, idx_packed, zeros)
    h, oacc = _tc_mid(h, p, W0, b0.reshape(1, D), oW[1], oacc)
    p = _sc_segment_sum(h, idx_packed, zeros)
    h, oacc = _tc_mid(h, p, W1, b1.reshape(1, D), oW[2], oacc)
    p = _sc_segment_sum(h, idx_packed, zeros)
    return _tc_fin(h, p, W2, b2.reshape(1, D), oW[3], oacc,
                   out_b.reshape(1, D))


# speedup vs baseline: 1.0221x; 1.0221x over previous
import functools

import jax
import jax.numpy as jnp
from jax import lax
from jax.experimental import pallas as pl
from jax.experimental.pallas import tpu as pltpu
from jax.experimental.pallas import tpu_sc as plsc

N_NODES = 10000
N_EDGES = 320000
D = 128

NC = 2
NS = 16
NW = NC * NS
EPW = N_EDGES // NW
CHUNK = 40
EPW_PAD = 10080
NCHUNK = EPW_PAD // CHUNK
ACC_ROWS = N_NODES + 32
RPT = 624
RTAIL = N_NODES - NS * RPT

_ROW_BLK = 1000


NBUF = 6
NBUF_I = 12
LOOK = 4
LOOK_I = 8
UNROLL = 12


def _sc_seg_body(h_hbm, idx_hbm, z_hbm, out_hbm,
                 idx_v, rows_v, sem_i, sem_g, sem_s, acc):
    c = lax.axis_index("c")
    s = lax.axis_index("s")
    wid = s * NC + c

    def rows(b):
        return rows_v.at[pl.ds(b * CHUNK, CHUNK)]

    def start_idx(j, bi):
        pltpu.async_copy(idx_hbm.at[wid, j], idx_v.at[bi], sem_i.at[bi])

    def wait_idx(j, bi):
        pltpu.make_async_copy(idx_hbm.at[wid, j], idx_v.at[bi],
                              sem_i.at[bi]).wait()

    def start_gather(j, b, bi):
        pltpu.async_copy(h_hbm.at[idx_v.at[bi, 0]], rows(b), sem_g.at[b])

    def wait_gather(j, b, bi):
        pltpu.make_async_copy(h_hbm.at[idx_v.at[bi, 0]], rows(b),
                              sem_g.at[b]).wait()

    def start_scatter(j, b, bi):
        pltpu.async_copy(rows(b), acc.at[idx_v.at[bi, 1]], sem_s.at[b],
                         add=True)

    def wait_scatter(j, b, bi):
        pltpu.make_async_copy(rows(b), acc.at[idx_v.at[bi, 1]],
                              sem_s.at[b]).wait()

    pltpu.sync_copy(z_hbm.at[pl.ds(s * RPT, RPT)],
                    acc.at[pl.ds(s * RPT, RPT)])

    @pl.when(s == 0)
    def _():
        pltpu.sync_copy(z_hbm.at[pl.ds(NS * RPT, RTAIL)],
                        acc.at[pl.ds(NS * RPT, RTAIL)])
    plsc.subcore_barrier()

    for j in range(LOOK_I):
        start_idx(j, j % NBUF_I)
    for j in range(LOOK):
        wait_idx(j, j % NBUF_I)
        start_gather(j, j % NBUF, j % NBUF_I)

    def outer(o, carry):
        j0 = o * UNROLL
        for u in range(UNROLL):
            j = j0 + u
            b, bi = u % NBUF, u % NBUF_I
            bg, big = (u + LOOK) % NBUF, (u + LOOK) % NBUF_I
            ji, jg = j + LOOK_I, j + LOOK

            @pl.when(ji < NCHUNK)
            def _():
                start_idx(ji, (u + LOOK_I) % NBUF_I)

            @pl.when(jg < NCHUNK)
            def _():
                @pl.when(jg >= NBUF)
                def _():
                    wait_scatter(jg - NBUF, bg, (u + LOOK - NBUF) % NBUF_I)
                wait_idx(jg, big)
                start_gather(jg, bg, big)

            wait_gather(j, b, bi)
            start_scatter(j, b, bi)
        return carry

    lax.fori_loop(0, NCHUNK // UNROLL, outer, 0)
    for j in range(NCHUNK - NBUF, NCHUNK):
        wait_scatter(j, j % NBUF, j % NBUF_I)
    plsc.subcore_barrier()
    pltpu.sync_copy(acc.at[pl.ds(s * RPT, RPT)],
                    out_hbm.at[c, pl.ds(s * RPT, RPT)])

    @pl.when(s == 0)
    def _():
        pltpu.sync_copy(acc.at[pl.ds(NS * RPT, RTAIL)],
                        out_hbm.at[c, pl.ds(NS * RPT, RTAIL)])


def _sc_segment_sum(h, idx_packed, zeros):
    mesh = plsc.VectorSubcoreMesh(core_axis_name="c", subcore_axis_name="s")
    k = pl.kernel(
        _sc_seg_body,
        out_type=jax.ShapeDtypeStruct((NC, N_NODES, D), jnp.float32),
        mesh=mesh,
        scratch_types=[
            pltpu.VMEM((NBUF_I, 2, CHUNK), jnp.int32),
            pltpu.VMEM((NBUF * CHUNK, D), jnp.float32),
            pltpu.SemaphoreType.DMA((NBUF_I,)),
            pltpu.SemaphoreType.DMA((NBUF,)),
            pltpu.SemaphoreType.DMA((NBUF,)),
            pltpu.VMEM_SHARED((ACC_ROWS, D), jnp.float32),
        ],
    )
    return k(h, idx_packed, zeros)


_HSPEC = pl.BlockSpec((_ROW_BLK, D), lambda i: (i, 0))
_WSPEC = pl.BlockSpec((D, D), lambda i: (0, 0))
_BSPEC = pl.BlockSpec((1, D), lambda i: (0, 0))
_PSPEC = pl.BlockSpec((NC, _ROW_BLK, D), lambda i: (0, i, 0))
_2OUT = [jax.ShapeDtypeStruct((N_NODES, D), jnp.float32)] * 2


def _in_body(x_ref, w_ref, b_ref, ow_ref, h_ref, oacc_ref):
    h = (jnp.dot(x_ref[...], w_ref[...],
                 preferred_element_type=jnp.float32) + b_ref[...])
    h_ref[...] = h
    oacc_ref[...] = jnp.dot(h, ow_ref[...], preferred_element_type=jnp.float32)


def _tc_in(x, W, b, oW):
    return pl.pallas_call(
        _in_body,
        grid=(N_NODES // _ROW_BLK,),
        in_specs=[_HSPEC, _WSPEC, _BSPEC, _WSPEC],
        out_specs=[_HSPEC, _HSPEC],
        out_shape=_2OUT,
    )(x, W, b, oW)


def _crit_body(h_ref, p_ref, w_ref, b_ref, h_ref_o):
    acc = h_ref[...] + p_ref[0] + p_ref[1]
    h_ref_o[...] = (jnp.dot(acc, w_ref[...],
                            preferred_element_type=jnp.float32) + b_ref[...])


def _tc_crit(h, p, W, b):
    return pl.pallas_call(
        _crit_body,
        grid=(N_NODES // _ROW_BLK,),
        in_specs=[_HSPEC, _PSPEC, _WSPEC, _BSPEC],
        out_specs=_HSPEC,
        out_shape=jax.ShapeDtypeStruct((N_NODES, D), jnp.float32),
    )(h, p, W, b)


def _side_body(oacc_in_ref, h_ref, ow_ref, oacc_ref):
    oacc_ref[...] = oacc_in_ref[...] + jnp.dot(
        h_ref[...], ow_ref[...], preferred_element_type=jnp.float32)


def _tc_side(oacc, h, oW):
    return pl.pallas_call(
        _side_body,
        grid=(N_NODES // _ROW_BLK,),
        in_specs=[_HSPEC, _HSPEC, _WSPEC],
        out_specs=_HSPEC,
        out_shape=jax.ShapeDtypeStruct((N_NODES, D), jnp.float32),
    )(oacc, h, oW)


def _fin_body(h_ref, p_ref, w_ref, b_ref, ow_ref, oacc_in_ref, ob_ref, o_ref):
    acc = h_ref[...] + p_ref[0] + p_ref[1]
    hn = (jnp.dot(acc, w_ref[...],
                  preferred_element_type=jnp.float32) + b_ref[...])
    o_ref[...] = (oacc_in_ref[...] + jnp.dot(
        hn, ow_ref[...], preferred_element_type=jnp.float32) + ob_ref[...])


def _tc_fin(h, p, W, b, oW, oacc, ob):
    return pl.pallas_call(
        _fin_body,
        grid=(N_NODES // _ROW_BLK,),
        in_specs=[_HSPEC, _PSPEC, _WSPEC, _BSPEC, _WSPEC, _HSPEC, _BSPEC],
        out_specs=_HSPEC,
        out_shape=jax.ShapeDtypeStruct((N_NODES, D), jnp.float32),
    )(h, p, W, b, oW, oacc, ob)


def kernel(x, edge_index, in_W, in_b, W0, b0, W1, b1, W2, b2, out_W, out_b):
    src = edge_index[0].astype(jnp.int32).reshape(NW, EPW)
    dst = edge_index[1].astype(jnp.int32).reshape(NW, EPW)
    npad = EPW_PAD - EPW
    pad_src = jnp.broadcast_to((jnp.arange(npad, dtype=jnp.int32) * 125)
                               % N_NODES, (NW, npad))
    pad_dst = jnp.broadcast_to(N_NODES + (jnp.arange(npad, dtype=jnp.int32)
                                          % 32), (NW, npad))
    src = jnp.concatenate([src, pad_src], axis=1)
    dst = jnp.concatenate([dst, pad_dst], axis=1)
    src = src.reshape(NW, NCHUNK, 1, CHUNK)
    dst = dst.reshape(NW, NCHUNK, 1, CHUNK)
    idx_packed = jnp.concatenate([src, dst], axis=2)
    zeros = jnp.zeros((N_NODES, D), jnp.float32)
    oW = [out_W[i * D:(i + 1) * D] for i in range(4)]

    h0, oacc = _tc_in(x, in_W, in_b.reshape(1, D), oW[0])
    p = _sc_segment_sum(h0, idx_packed, zeros)
    h1 = _tc_crit(h0, p, W0, b0.reshape(1, D))
    p = _sc_segment_sum(h1, idx_packed, zeros)
    oacc = _tc_side(oacc, h1, oW[1])
    h2 = _tc_crit(h1, p, W1, b1.reshape(1, D))
    p = _sc_segment_sum(h2, idx_packed, zeros)
    oacc = _tc_side(oacc, h2, oW[2])
    return _tc_fin(h2, p, W2, b2.reshape(1, D), oW[3], oacc,
                   out_b.reshape(1, D))


# speedup vs baseline: 1.0398x; 1.0172x over previous
import functools

import jax
import jax.numpy as jnp
from jax import lax
from jax.experimental import pallas as pl
from jax.experimental.pallas import tpu as pltpu
from jax.experimental.pallas import tpu_sc as plsc

N_NODES = 10000
N_EDGES = 320000
D = 128

NC = 2
NS = 16
NW = NC * NS
EPW = N_EDGES // NW
CHUNK = 40
EPW_PAD = 10080
NCHUNK = EPW_PAD // CHUNK
ACC_ROWS = N_NODES + 32
RPT = 624
RTAIL = N_NODES - NS * RPT

_ROW_BLK = 1000


NBUF = 6
NBUF_I = 12
LOOK = 4
LOOK_I = 8
UNROLL = 12


def _sc_seg_body(h_hbm, idx_hbm, z_hbm, out_hbm,
                 idx_v, rows_v, sem_i, sem_g, sem_s, sem_z, acc):
    c = lax.axis_index("c")
    s = lax.axis_index("s")
    wid = s * NC + c

    def rows(b):
        return rows_v.at[pl.ds(b * CHUNK, CHUNK)]

    def start_idx(j, bi):
        pltpu.async_copy(idx_hbm.at[wid, j], idx_v.at[bi], sem_i.at[bi])

    def wait_idx(j, bi):
        pltpu.make_async_copy(idx_hbm.at[wid, j], idx_v.at[bi],
                              sem_i.at[bi]).wait()

    def start_gather(j, b, bi):
        pltpu.async_copy(h_hbm.at[idx_v.at[bi, 0]], rows(b), sem_g.at[b])

    def wait_gather(j, b, bi):
        pltpu.make_async_copy(h_hbm.at[idx_v.at[bi, 0]], rows(b),
                              sem_g.at[b]).wait()

    def start_scatter(j, b, bi):
        pltpu.async_copy(rows(b), acc.at[idx_v.at[bi, 1]], sem_s.at[b],
                         add=True)

    def wait_scatter(j, b, bi):
        pltpu.make_async_copy(rows(b), acc.at[idx_v.at[bi, 1]],
                              sem_s.at[b]).wait()

    pltpu.async_copy(z_hbm.at[pl.ds(s * RPT, RPT)],
                     acc.at[pl.ds(s * RPT, RPT)], sem_z)

    @pl.when(s == 0)
    def _():
        pltpu.async_copy(z_hbm.at[pl.ds(NS * RPT, RTAIL)],
                         acc.at[pl.ds(NS * RPT, RTAIL)], sem_z)

    for j in range(LOOK_I):
        start_idx(j, j % NBUF_I)
    for j in range(LOOK):
        wait_idx(j, j % NBUF_I)
        start_gather(j, j % NBUF, j % NBUF_I)

    pltpu.make_async_copy(z_hbm.at[pl.ds(s * RPT, RPT)],
                          acc.at[pl.ds(s * RPT, RPT)], sem_z).wait()

    @pl.when(s == 0)
    def _():
        pltpu.make_async_copy(z_hbm.at[pl.ds(NS * RPT, RTAIL)],
                              acc.at[pl.ds(NS * RPT, RTAIL)], sem_z).wait()
    plsc.subcore_barrier()

    def outer(o, carry):
        j0 = o * UNROLL
        for u in range(UNROLL):
            j = j0 + u
            b, bi = u % NBUF, u % NBUF_I
            bg, big = (u + LOOK) % NBUF, (u + LOOK) % NBUF_I
            ji, jg = j + LOOK_I, j + LOOK

            @pl.when(ji < NCHUNK)
            def _():
                start_idx(ji, (u + LOOK_I) % NBUF_I)

            @pl.when(jg < NCHUNK)
            def _():
                @pl.when(jg >= NBUF)
                def _():
                    wait_scatter(jg - NBUF, bg, (u + LOOK - NBUF) % NBUF_I)
                wait_idx(jg, big)
                start_gather(jg, bg, big)

            wait_gather(j, b, bi)
            start_scatter(j, b, bi)
        return carry

    lax.fori_loop(0, NCHUNK // UNROLL, outer, 0)
    for j in range(NCHUNK - NBUF, NCHUNK):
        wait_scatter(j, j % NBUF, j % NBUF_I)
    plsc.subcore_barrier()
    pltpu.sync_copy(acc.at[pl.ds(s * RPT, RPT)],
                    out_hbm.at[c, pl.ds(s * RPT, RPT)])

    @pl.when(s == 0)
    def _():
        pltpu.sync_copy(acc.at[pl.ds(NS * RPT, RTAIL)],
                        out_hbm.at[c, pl.ds(NS * RPT, RTAIL)])


def _sc_segment_sum(h, idx_packed, zeros):
    mesh = plsc.VectorSubcoreMesh(core_axis_name="c", subcore_axis_name="s")
    k = pl.kernel(
        _sc_seg_body,
        out_type=jax.ShapeDtypeStruct((NC, N_NODES, D), jnp.float32),
        mesh=mesh,
        scratch_types=[
            pltpu.VMEM((NBUF_I, 2, CHUNK), jnp.int32),
            pltpu.VMEM((NBUF * CHUNK, D), jnp.float32),
            pltpu.SemaphoreType.DMA((NBUF_I,)),
            pltpu.SemaphoreType.DMA((NBUF,)),
            pltpu.SemaphoreType.DMA((NBUF,)),
            pltpu.SemaphoreType.DMA,
            pltpu.VMEM_SHARED((ACC_ROWS, D), jnp.float32),
        ],
    )
    return k(h, idx_packed, zeros)


_HSPEC = pl.BlockSpec((_ROW_BLK, D), lambda i: (i, 0))
_WSPEC = pl.BlockSpec((D, D), lambda i: (0, 0))
_BSPEC = pl.BlockSpec((1, D), lambda i: (0, 0))
_PSPEC = pl.BlockSpec((NC, _ROW_BLK, D), lambda i: (0, i, 0))
_2OUT = [jax.ShapeDtypeStruct((N_NODES, D), jnp.float32)] * 2


def _in_body(x_ref, w_ref, b_ref, ow_ref, h_ref, oacc_ref):
    h = (jnp.dot(x_ref[...], w_ref[...],
                 preferred_element_type=jnp.float32) + b_ref[...])
    h_ref[...] = h
    oacc_ref[...] = jnp.dot(h, ow_ref[...], preferred_element_type=jnp.float32)


def _tc_in(x, W, b, oW):
    return pl.pallas_call(
        _in_body,
        grid=(N_NODES // _ROW_BLK,),
        in_specs=[_HSPEC, _WSPEC, _BSPEC, _WSPEC],
        out_specs=[_HSPEC, _HSPEC],
        out_shape=_2OUT,
    )(x, W, b, oW)


def _crit_body(h_ref, p_ref, w_ref, b_ref, h_ref_o):
    acc = h_ref[...] + p_ref[0] + p_ref[1]
    h_ref_o[...] = (jnp.dot(acc, w_ref[...],
                            preferred_element_type=jnp.float32) + b_ref[...])


def _tc_crit(h, p, W, b):
    return pl.pallas_call(
        _crit_body,
        grid=(N_NODES // _ROW_BLK,),
        in_specs=[_HSPEC, _PSPEC, _WSPEC, _BSPEC],
        out_specs=_HSPEC,
        out_shape=jax.ShapeDtypeStruct((N_NODES, D), jnp.float32),
    )(h, p, W, b)


def _side_body(oacc_in_ref, h_ref, ow_ref, oacc_ref):
    oacc_ref[...] = oacc_in_ref[...] + jnp.dot(
        h_ref[...], ow_ref[...], preferred_element_type=jnp.float32)


def _tc_side(oacc, h, oW):
    return pl.pallas_call(
        _side_body,
        grid=(N_NODES // _ROW_BLK,),
        in_specs=[_HSPEC, _HSPEC, _WSPEC],
        out_specs=_HSPEC,
        out_shape=jax.ShapeDtypeStruct((N_NODES, D), jnp.float32),
    )(oacc, h, oW)


def _fin_body(h_ref, p_ref, w_ref, b_ref, ow_ref, oacc_in_ref, ob_ref, o_ref):
    acc = h_ref[...] + p_ref[0] + p_ref[1]
    hn = (jnp.dot(acc, w_ref[...],
                  preferred_element_type=jnp.float32) + b_ref[...])
    o_ref[...] = (oacc_in_ref[...] + jnp.dot(
        hn, ow_ref[...], preferred_element_type=jnp.float32) + ob_ref[...])


def _tc_fin(h, p, W, b, oW, oacc, ob):
    return pl.pallas_call(
        _fin_body,
        grid=(N_NODES // _ROW_BLK,),
        in_specs=[_HSPEC, _PSPEC, _WSPEC, _BSPEC, _WSPEC, _HSPEC, _BSPEC],
        out_specs=_HSPEC,
        out_shape=jax.ShapeDtypeStruct((N_NODES, D), jnp.float32),
    )(h, p, W, b, oW, oacc, ob)


def kernel(x, edge_index, in_W, in_b, W0, b0, W1, b1, W2, b2, out_W, out_b):
    src = edge_index[0].astype(jnp.int32).reshape(NW, EPW)
    dst = edge_index[1].astype(jnp.int32).reshape(NW, EPW)
    npad = EPW_PAD - EPW
    pad_src = jnp.broadcast_to((jnp.arange(npad, dtype=jnp.int32) * 125)
                               % N_NODES, (NW, npad))
    pad_dst = jnp.broadcast_to(N_NODES + (jnp.arange(npad, dtype=jnp.int32)
                                          % 32), (NW, npad))
    src = jnp.concatenate([src, pad_src], axis=1)
    dst = jnp.concatenate([dst, pad_dst], axis=1)
    src = src.reshape(NW, NCHUNK, 1, CHUNK)
    dst = dst.reshape(NW, NCHUNK, 1, CHUNK)
    idx_packed = jnp.concatenate([src, dst], axis=2)
    zeros = jnp.zeros((N_NODES, D), jnp.float32)
    oW = [out_W[i * D:(i + 1) * D] for i in range(4)]

    h0, oacc = _tc_in(x, in_W, in_b.reshape(1, D), oW[0])
    p = _sc_segment_sum(h0, idx_packed, zeros)
    h1 = _tc_crit(h0, p, W0, b0.reshape(1, D))
    p = _sc_segment_sum(h1, idx_packed, zeros)
    oacc = _tc_side(oacc, h1, oW[1])
    h2 = _tc_crit(h1, p, W1, b1.reshape(1, D))
    p = _sc_segment_sum(h2, idx_packed, zeros)
    oacc = _tc_side(oacc, h2, oW[2])
    return _tc_fin(h2, p, W2, b2.reshape(1, D), oW[3], oacc,
                   out_b.reshape(1, D))
